# Initial kernel scaffold; baseline (speedup 1.0000x reference)
import functools

import jax
import jax.numpy as jnp
from jax import lax
from jax.experimental import pallas as pl
from jax.experimental.pallas import tpu as pltpu
from jax.experimental.pallas import tpu_sc as plsc

N = 10000
NP = 10240
D = 128
E = 320000
NC = 2
NS = 16
CHUNK = 128
CHUNKS = 82
E_PAD = NC * NS * CHUNKS * CHUNK
RPT = NP // NS

_mesh = plsc.VectorSubcoreMesh(core_axis_name="c", subcore_axis_name="s")


def _sc_deg_body(dstd_hbm, ones_hbm, zeros_hbm, out_hbm, dst_v, ones_v, deg_sh):
    c = lax.axis_index("c")
    s = lax.axis_index("s")
    pltpu.sync_copy(zeros_hbm.at[pl.ds(s * RPT, RPT)],
                    deg_sh.at[pl.ds(s * RPT, RPT)])
    pltpu.sync_copy(dstd_hbm.at[c, s], dst_v)
    pltpu.sync_copy(ones_hbm, ones_v)
    plsc.subcore_barrier()

    def body(j, carry):
        pltpu.sync_copy(ones_v, deg_sh.at[dst_v.at[j]], add=True)
        return carry

    lax.fori_loop(0, CHUNKS, body, 0)
    plsc.subcore_barrier()
    pltpu.sync_copy(deg_sh.at[pl.ds(s * RPT, RPT)],
                    out_hbm.at[c, pl.ds(s * RPT, RPT)])


_sc_deg = functools.partial(
    pl.kernel,
    out_type=jax.ShapeDtypeStruct((NC, NP, 16), jnp.float32),
    mesh=_mesh,
    scratch_types=[
        pltpu.VMEM((CHUNKS, CHUNK), jnp.int32),
        pltpu.VMEM((CHUNK, 16), jnp.float32),
        pltpu.VMEM_SHARED((NP, 16), jnp.float32),
    ],
)(_sc_deg_body)


def _sc_agg_body(h_hbm, srcd_hbm, dstd_hbm, zeros_hbm, out_hbm,
                 src_v, dst_v, rows_v, acc_sh, sem):
    c = lax.axis_index("c")
    s = lax.axis_index("s")
    pltpu.sync_copy(zeros_hbm.at[pl.ds(s * RPT, RPT)],
                    acc_sh.at[pl.ds(s * RPT, RPT)])
    pltpu.sync_copy(srcd_hbm.at[c, s], src_v)
    pltpu.sync_copy(dstd_hbm.at[c, s], dst_v)
    plsc.subcore_barrier()

    def body(j, carry):
        pltpu.async_copy(h_hbm.at[src_v.at[j]], rows_v, sem).wait()
        pltpu.sync_copy(rows_v, acc_sh.at[dst_v.at[j]], add=True)
        return carry

    lax.fori_loop(0, CHUNKS, body, 0)
    plsc.subcore_barrier()
    pltpu.sync_copy(acc_sh.at[pl.ds(s * RPT, RPT)],
                    out_hbm.at[c, pl.ds(s * RPT, RPT)])


_sc_agg = functools.partial(
    pl.kernel,
    out_type=jax.ShapeDtypeStruct((NC, NP, D), jnp.float32),
    mesh=_mesh,
    scratch_types=[
        pltpu.VMEM((CHUNKS, CHUNK), jnp.int32),
        pltpu.VMEM((CHUNKS, CHUNK), jnp.int32),
        pltpu.VMEM((CHUNK, D), jnp.float32),
        pltpu.VMEM_SHARED((NP, D), jnp.float32),
        pltpu.SemaphoreType.DMA,
    ],
)(_sc_agg_body)


BR = 2048


def _dinv_of(deg_ref):
    deg = deg_ref[0, :, 0] + deg_ref[1, :, 0]
    return jnp.where(deg > 0, lax.rsqrt(deg), 0.0)


def _tc_pre_body(x_ref, w_ref, deg_ref, o_ref):
    dinv = _dinv_of(deg_ref)
    h = jnp.dot(x_ref[...], w_ref[...], preferred_element_type=jnp.float32)
    o_ref[...] = h * dinv[:, None]


def _tc_mid_body(acc_ref, deg_ref, b_ref, w_ref, o_ref):
    dinv = _dinv_of(deg_ref)
    t = (acc_ref[0] + acc_ref[1]) * dinv[:, None] + b_ref[...]
    r = jnp.maximum(t, 0.0)
    o_ref[...] = jnp.dot(r, w_ref[...],
                         preferred_element_type=jnp.float32) * dinv[:, None]


def _tc_post_body(acc_ref, deg_ref, b_ref, o_ref):
    dinv = _dinv_of(deg_ref)
    o_ref[...] = (acc_ref[0] + acc_ref[1]) * dinv[:, None] + b_ref[...]


_deg_spec = pl.BlockSpec((NC, BR, 16), lambda i: (0, i, 0))
_acc_spec = pl.BlockSpec((NC, BR, D), lambda i: (0, i, 0))
_row_spec = pl.BlockSpec((BR, D), lambda i: (i, 0))
_w_spec = pl.BlockSpec((D, D), lambda i: (0, 0))
_b_spec = pl.BlockSpec((1, D), lambda i: (0, 0))

_tc_pre = pl.pallas_call(
    _tc_pre_body,
    grid=(NP // BR,),
    in_specs=[_row_spec, _w_spec, _deg_spec],
    out_specs=_row_spec,
    out_shape=jax.ShapeDtypeStruct((NP, D), jnp.float32),
)

_tc_mid = pl.pallas_call(
    _tc_mid_body,
    grid=(NP // BR,),
    in_specs=[_acc_spec, _deg_spec, _b_spec, _w_spec],
    out_specs=_row_spec,
    out_shape=jax.ShapeDtypeStruct((NP, D), jnp.float32),
)

_tc_post = pl.pallas_call(
    _tc_post_body,
    grid=(NP // BR,),
    in_specs=[_acc_spec, _deg_spec, _b_spec],
    out_specs=_row_spec,
    out_shape=jax.ShapeDtypeStruct((NP, D), jnp.float32),
)


def kernel(x, edge_index, W1, b1, W2, b2):
    loop = jnp.arange(N, dtype=jnp.int32)
    pad = E_PAD - (E + N)
    src = jnp.concatenate([
        edge_index[0].astype(jnp.int32), loop,
        jnp.zeros((pad,), jnp.int32),
    ]).reshape(NC, NS, CHUNKS, CHUNK)
    dst = jnp.concatenate([
        edge_index[1].astype(jnp.int32), loop,
        jnp.full((pad,), N, jnp.int32),
    ]).reshape(NC, NS, CHUNKS, CHUNK)

    x_pad = jnp.pad(x, ((0, NP - N), (0, 0)))
    zeros_d = jnp.zeros((NP, D), jnp.float32)
    zeros_16 = jnp.zeros((NP, 16), jnp.float32)
    ones_16 = jnp.ones((CHUNK, 16), jnp.float32)
    b1r = b1.reshape(1, D)
    b2r = b2.reshape(1, D)

    deg_parts = _sc_deg(dst, ones_16, zeros_16)
    h1 = _tc_pre(x_pad, W1, deg_parts)
    acc1 = _sc_agg(h1, src, dst, zeros_d)
    h2 = _tc_mid(acc1, deg_parts, b1r, W2)
    acc2 = _sc_agg(h2, src, dst, zeros_d)
    out = _tc_post(acc2, deg_parts, b2r)
    return out[:N]


# speedup vs baseline: 7.1478x; 7.1478x over previous
import functools

import jax
import jax.numpy as jnp
from jax import lax
from jax.experimental import pallas as pl
from jax.experimental.pallas import tpu as pltpu
from jax.experimental.pallas import tpu_sc as plsc

N = 10000
NP = 10240
D = 128
E = 320000
NC = 2
NS = 16
CHUNK = 128
CHUNKS = 82
E_PAD = NC * NS * CHUNKS * CHUNK
RPT = NP // NS

_mesh = plsc.VectorSubcoreMesh(core_axis_name="c", subcore_axis_name="s")


def _sc_agg_body(h_hbm, srcd_hbm, dstd_hbm, zeros_hbm, out_hbm,
                 src_v, dst_v, rows_v, acc_sh, sem):
    c = lax.axis_index("c")
    s = lax.axis_index("s")
    pltpu.sync_copy(zeros_hbm.at[pl.ds(s * RPT, RPT)],
                    acc_sh.at[pl.ds(s * RPT, RPT)])
    pltpu.sync_copy(srcd_hbm.at[c, s], src_v)
    pltpu.sync_copy(dstd_hbm.at[c, s], dst_v)
    plsc.subcore_barrier()

    def body(j, carry):
        pltpu.async_copy(h_hbm.at[src_v.at[j]], rows_v, sem).wait()
        pltpu.sync_copy(rows_v, acc_sh.at[dst_v.at[j]], add=True)
        return carry

    lax.fori_loop(0, CHUNKS, body, 0)
    plsc.subcore_barrier()
    pltpu.sync_copy(acc_sh.at[pl.ds(s * RPT, RPT)],
                    out_hbm.at[c, pl.ds(s * RPT, RPT)])


_sc_agg = functools.partial(
    pl.kernel,
    out_type=jax.ShapeDtypeStruct((NC, NP, D), jnp.float32),
    mesh=_mesh,
    scratch_types=[
        pltpu.VMEM((CHUNKS, CHUNK), jnp.int32),
        pltpu.VMEM((CHUNKS, CHUNK), jnp.int32),
        pltpu.VMEM((CHUNK, D), jnp.float32),
        pltpu.VMEM_SHARED((NP, D), jnp.float32),
        pltpu.SemaphoreType.DMA,
    ],
)(_sc_agg_body)


BR = 2048


def _dinv_of(deg_ref):
    deg = deg_ref[0, :, 0] + deg_ref[1, :, 0]
    return jnp.where(deg > 0, lax.rsqrt(deg), 0.0)


def _tc_pre_body(x_ref, w_ref, deg_ref, o_ref):
    dinv = _dinv_of(deg_ref)
    h = jnp.dot(x_ref[...], w_ref[...], preferred_element_type=jnp.float32)
    o_ref[...] = h * dinv[:, None]


def _tc_mid_body(acc_ref, deg_ref, b_ref, w_ref, o_ref):
    dinv = _dinv_of(deg_ref)
    t = (acc_ref[0] + acc_ref[1]) * dinv[:, None] + b_ref[...]
    r = jnp.maximum(t, 0.0)
    o_ref[...] = jnp.dot(r, w_ref[...],
                         preferred_element_type=jnp.float32) * dinv[:, None]


def _tc_post_body(acc_ref, deg_ref, b_ref, o_ref):
    dinv = _dinv_of(deg_ref)
    o_ref[...] = (acc_ref[0] + acc_ref[1]) * dinv[:, None] + b_ref[...]


_acc_spec = pl.BlockSpec((NC, BR, D), lambda i: (0, i, 0))
_deg_spec = _acc_spec
_row_spec = pl.BlockSpec((BR, D), lambda i: (i, 0))
_w_spec = pl.BlockSpec((D, D), lambda i: (0, 0))
_b_spec = pl.BlockSpec((1, D), lambda i: (0, 0))

_tc_pre = pl.pallas_call(
    _tc_pre_body,
    grid=(NP // BR,),
    in_specs=[_row_spec, _w_spec, _deg_spec],
    out_specs=_row_spec,
    out_shape=jax.ShapeDtypeStruct((NP, D), jnp.float32),
)

_tc_mid = pl.pallas_call(
    _tc_mid_body,
    grid=(NP // BR,),
    in_specs=[_acc_spec, _deg_spec, _b_spec, _w_spec],
    out_specs=_row_spec,
    out_shape=jax.ShapeDtypeStruct((NP, D), jnp.float32),
)

_tc_post = pl.pallas_call(
    _tc_post_body,
    grid=(NP // BR,),
    in_specs=[_acc_spec, _deg_spec, _b_spec],
    out_specs=_row_spec,
    out_shape=jax.ShapeDtypeStruct((NP, D), jnp.float32),
)


def kernel(x, edge_index, W1, b1, W2, b2):
    loop = jnp.arange(N, dtype=jnp.int32)
    pad = E_PAD - (E + N)
    src = jnp.concatenate([
        edge_index[0].astype(jnp.int32), loop,
        jnp.zeros((pad,), jnp.int32),
    ]).reshape(NC, NS, CHUNKS, CHUNK)
    dst = jnp.concatenate([
        edge_index[1].astype(jnp.int32), loop,
        jnp.full((pad,), N, jnp.int32),
    ]).reshape(NC, NS, CHUNKS, CHUNK)

    x_pad = jnp.pad(x, ((0, NP - N), (0, 0)))
    zeros_d = jnp.zeros((NP, D), jnp.float32)
    ones_d = jnp.ones((NP, D), jnp.float32)
    b1r = b1.reshape(1, D)
    b2r = b2.reshape(1, D)

    deg_parts = _sc_agg(ones_d, src, dst, zeros_d)
    h1 = _tc_pre(x_pad, W1, deg_parts)
    acc1 = _sc_agg(h1, src, dst, zeros_d)
    h2 = _tc_mid(acc1, deg_parts, b1r, W2)
    acc2 = _sc_agg(h2, src, dst, zeros_d)
    out = _tc_post(acc2, deg_parts, b2r)
    return out[:N]
